# baseline (speedup 1.0000x reference)
import functools

import jax
import jax.numpy as jnp
from jax.experimental import pallas as pl
from jax.experimental.pallas import tpu as pltpu

_C = 128
_S = 56
_RS = 64
_F = _S * _RS
_OFF = 256
_W = 4096


def _dn_kernel(x_ref, th_ref, p_ref, sig_ref, a_ref, nIT_ref, nU_ref,
               bias_ref, out_ref):
    f32 = jnp.float32
    t_idx = jax.lax.broadcasted_iota(jnp.int32, (36, 1), 0)
    xv = -3.0 + 1.2 * (t_idx // 6).astype(f32)
    yv = -3.0 + 1.2 * (t_idx % 6).astype(f32)
    th = th_ref[0]
    pr = p_ref[0]
    sg = sig_ref[0]
    ar = a_ref[0]
    ct = jnp.cos(th)
    st = jnp.sin(th)
    xrot = xv * ct + yv * st
    yrot = yv * ct - xv * st
    inv_p2 = 1.0 / (pr * pr)
    inv_s2 = 1.0 / (sg * sg)
    amp = ar / (2.0 * jnp.pi * pr * sg)
    g = amp * jnp.exp(-0.5 * (xrot * xrot * inv_p2 + yrot * yrot * inv_s2))

    nI_col = nIT_ref[0]
    nU_s = nU_ref[0, 0, 0]
    bias_s = bias_ref[0, 0, 0]
    bias_pow = jnp.exp(nU_s * jnp.log(bias_s))

    for b in range(x_ref.shape[0]):
        xb = x_ref[b, 0]
        lx = jnp.log(xb)
        xp = jnp.exp(nI_col * lx)
        y = jnp.dot(g.astype(jnp.bfloat16), xp.astype(jnp.bfloat16),
                    preferred_element_type=f32)
        acc = jnp.zeros((1, _F), f32)
        for t in range(36):
            dy, dx = t // 6, t % 6
            s = _OFF + (dy - 2) * _RS + (dx - 2)
            acc = acc + y[t:t + 1, s:s + _F]
        denom = bias_pow + acc
        num = jnp.exp(nU_s * lx[0:1, _OFF:_OFF + _F])
        out_ref[b, 0] = num / denom


@functools.partial(jax.jit, static_argnames=())
def kernel(x, theta, p, sig, a, nI, nU, bias):
    B = x.shape[0]
    C, S = _C, _S
    f32 = jnp.float32
    xw = jnp.pad(x.astype(f32), ((0, 0), (0, 0), (0, 0), (0, _RS - S)))
    xw = xw.reshape(B, C, 1, _F)
    xw = jnp.pad(xw, ((0, 0), (0, 0), (0, 0), (_OFF, _W - _F - _OFF)))
    row3 = lambda m: m.reshape(C, 1, C).astype(f32)
    scal3 = lambda v: v.reshape(C, 1, 1).astype(f32)
    pair_spec = pl.BlockSpec((1, 1, C), lambda i: (i, 0, 0))
    scal_spec = pl.BlockSpec((1, 1, 1), lambda i: (i, 0, 0))
    out = pl.pallas_call(
        _dn_kernel,
        grid=(C,),
        in_specs=[pl.BlockSpec((B, 1, 1, _W), lambda i: (0, i, 0, 0)),
                  pair_spec, pair_spec, pair_spec, pair_spec,
                  pl.BlockSpec((1, C, 1), lambda i: (i, 0, 0)),
                  scal_spec, scal_spec],
        out_specs=pl.BlockSpec((B, 1, 1, _F), lambda i: (0, i, 0, 0)),
        out_shape=jax.ShapeDtypeStruct((B, C, 1, _F), f32),
        compiler_params=pltpu.CompilerParams(
            dimension_semantics=("parallel",)),
    )(xw, row3(theta), row3(p), row3(sig), row3(a),
      nI.astype(f32).reshape(C, C, 1), scal3(nU), scal3(bias))
    return out.reshape(B, C, S, _RS)[:, :, :, :S]


# speedup vs baseline: 1.0345x; 1.0345x over previous
import functools

import jax
import jax.numpy as jnp
from jax.experimental import pallas as pl
from jax.experimental.pallas import tpu as pltpu

_C = 128
_S = 56
_RS = 64
_F = _S * _RS
_OFF = 256
_W = 4096


def _dn_kernel(x_ref, th_ref, p_ref, sig_ref, a_ref, nIT_ref, nU_ref,
               bias_ref, out_ref):
    f32 = jnp.float32
    t_idx = jax.lax.broadcasted_iota(jnp.int32, (36, 1), 0)
    xv = -3.0 + 1.2 * (t_idx // 6).astype(f32)
    yv = -3.0 + 1.2 * (t_idx % 6).astype(f32)
    th = th_ref[0]
    pr = p_ref[0]
    sg = sig_ref[0]
    ar = a_ref[0]
    ct = jnp.cos(th)
    st = jnp.sin(th)
    xrot = xv * ct + yv * st
    yrot = yv * ct - xv * st
    inv_p2 = 1.0 / (pr * pr)
    inv_s2 = 1.0 / (sg * sg)
    amp = ar / (2.0 * jnp.pi * pr * sg)
    g = amp * jnp.exp(-0.5 * (xrot * xrot * inv_p2 + yrot * yrot * inv_s2))

    nI_col = nIT_ref[0]
    nU_s = nU_ref[0, 0, 0]
    bias_s = bias_ref[0, 0, 0]
    bias_pow = jnp.exp(nU_s * jnp.log(bias_s))

    for b in range(x_ref.shape[0]):
        xb = x_ref[b, 0]
        lx = jnp.log(xb)
        xp = jnp.exp(nI_col * lx)
        y = jnp.dot(g, xp, preferred_element_type=f32)
        acc = jnp.zeros((1, _F), f32)
        for t in range(36):
            dy, dx = t // 6, t % 6
            s = _OFF + (dy - 2) * _RS + (dx - 2)
            acc = acc + y[t:t + 1, s:s + _F]
        denom = bias_pow + acc
        num = jnp.exp(nU_s * lx[0:1, _OFF:_OFF + _F])
        out_ref[b, 0] = num / denom


@functools.partial(jax.jit, static_argnames=())
def kernel(x, theta, p, sig, a, nI, nU, bias):
    B = x.shape[0]
    C, S = _C, _S
    f32 = jnp.float32
    xw = jnp.pad(x.astype(f32), ((0, 0), (0, 0), (4, 4), (0, _RS - S)))
    xw = xw.reshape(B, C, 1, _W)
    row3 = lambda m: m.reshape(C, 1, C).astype(f32)
    scal3 = lambda v: v.reshape(C, 1, 1).astype(f32)
    pair_spec = pl.BlockSpec((1, 1, C), lambda i: (i, 0, 0))
    scal_spec = pl.BlockSpec((1, 1, 1), lambda i: (i, 0, 0))
    out = pl.pallas_call(
        _dn_kernel,
        grid=(C,),
        in_specs=[pl.BlockSpec((B, 1, 1, _W), lambda i: (0, i, 0, 0)),
                  pair_spec, pair_spec, pair_spec, pair_spec,
                  pl.BlockSpec((1, C, 1), lambda i: (i, 0, 0)),
                  scal_spec, scal_spec],
        out_specs=pl.BlockSpec((B, 1, 1, _F), lambda i: (0, i, 0, 0)),
        out_shape=jax.ShapeDtypeStruct((B, C, 1, _F), f32),
        compiler_params=pltpu.CompilerParams(
            dimension_semantics=("parallel",)),
    )(xw, row3(theta), row3(p), row3(sig), row3(a),
      nI.astype(f32).reshape(C, C, 1), scal3(nU), scal3(bias))
    return out.reshape(B, C, S, _RS)[:, :, :, :S]
